# Initial kernel scaffold; baseline (speedup 1.0000x reference)
import jax
import jax.numpy as jnp
from jax.experimental import pallas as pl


def kernel(data, params):
    raise NotImplementedError("write your pallas kernel here")


# speedup vs baseline: 11.9090x; 11.9090x over previous
import functools

import jax
import jax.numpy as jnp
from jax.experimental import pallas as pl

B, T, N, C = 4, 16, 512, 7
GEOM, AUX, K = 3, 4, 20
FRAMES = B * T
OUT_D = 32
DM, NH, DH, FF, NC = 1024, 4, 256, 2048, 12

_F32 = jnp.float32
_IMAX = 2147483647


def _dot(a, b):
    return jax.lax.dot_general(a, b, (((1,), (0,)), ((), ())),
                               preferred_element_type=_F32)


def _dot_t(a, b):
    return jax.lax.dot_general(a, b, (((1,), (1,)), ((), ())),
                               preferred_element_type=_F32)


def _edge_layer_body(in_d, x_ref, aux_ref, wl_ref, wr_ref, eb1_ref,
                     ew2_ref, eb2_ref, a1a_ref, a1b_ref, ab1_ref,
                     aw2_ref, ab2_ref, lng_ref, lnb_ref, out_ref):
    g = x_ref[0]
    aux = aux_ref[0]

    gg = g * g
    sq_col = jnp.sum(gg, axis=1, keepdims=True)
    sq_row = _dot_t(jnp.ones((1, in_d), _F32), gg)
    d2 = jnp.maximum(sq_col + sq_row - 2.0 * _dot_t(g, g), 0.0)
    ii = jax.lax.broadcasted_iota(jnp.int32, (N, N), 0)
    jj = jax.lax.broadcasted_iota(jnp.int32, (N, N), 1)
    d2 = d2 + jnp.where(ii == jj, 1e10, 0.0).astype(_F32)
    key = jnp.bitwise_or(
        jnp.bitwise_and(jax.lax.bitcast_convert_type(d2, jnp.int32),
                        -512), jj)

    p = _dot(g, wl_ref[...]) + eb1_ref[...]
    q = _dot(g, wr_ref[...])
    at = _dot(aux, a1a_ref[...]) + ab1_ref[...]
    asrc = _dot(aux, a1b_ref[...])
    payload = jnp.concatenate([q, asrc], axis=1)

    acc = jnp.full((N, OUT_D), -jnp.inf, _F32)
    for _ in range(K):
        m = jnp.min(key, axis=1, keepdims=True)
        sel = key == m
        onehot = sel.astype(_F32)
        key = jnp.where(sel, _IMAX, key)
        gath = _dot(onehot, payload)
        h1 = jax.nn.relu(p + gath[:, :OUT_D])
        a1 = jax.nn.relu(at + gath[:, OUT_D:])
        h2 = jax.nn.relu(_dot(h1, ew2_ref[...]) + eb2_ref[...])
        gb = _dot(a1, aw2_ref[...]) + ab2_ref[...]
        mod = (jax.nn.sigmoid(gb[:, :OUT_D] + 1.0) * h2 + gb[:, OUT_D:])
        acc = jnp.maximum(acc, mod)

    mu = jnp.mean(acc, axis=1, keepdims=True)
    var = jnp.mean((acc - mu) * (acc - mu), axis=1, keepdims=True)
    y = (acc - mu) * jax.lax.rsqrt(var + 1e-5) * lng_ref[...] + lnb_ref[...]
    out_ref[0] = jax.nn.relu(y)


def _edge_layer(x, aux, lp):
    in_d = x.shape[-1]
    wl = lp['eW1'][:in_d] - lp['eW1'][in_d:]
    wr = lp['eW1'][in_d:]
    a1a = lp['aW1'][:AUX]
    a1b = lp['aW1'][AUX:]
    row = lambda v: v.reshape(1, -1)
    full = lambda a: pl.BlockSpec(a.shape, lambda f: (0,) * a.ndim)
    args = (aux, wl, wr, row(lp['eb1']), lp['eW2'], row(lp['eb2']),
            a1a, a1b, row(lp['ab1']), lp['aW2'], row(lp['ab2']),
            row(lp['ln_g']), row(lp['ln_b']))
    return pl.pallas_call(
        functools.partial(_edge_layer_body, in_d),
        grid=(FRAMES,),
        in_specs=[pl.BlockSpec((1, N, in_d), lambda f: (f, 0, 0)),
                  pl.BlockSpec((1, N, AUX), lambda f: (f, 0, 0))]
                 + [full(a) for a in args[1:]],
        out_specs=pl.BlockSpec((1, N, OUT_D), lambda f: (f, 0, 0)),
        out_shape=jax.ShapeDtypeStruct((FRAMES, N, OUT_D), _F32),
    )(x, *args)


def _pool_body(x1_ref, x2_ref, x3_ref, wa_ref, wb_ref, wc_ref, b_ref,
               out_ref):
    z = (_dot(x1_ref[0], wa_ref[...]) + _dot(x2_ref[0], wb_ref[...])
         + _dot(x3_ref[0], wc_ref[...]) + b_ref[...])
    z = jax.nn.relu(z)
    out_ref[0] = jnp.max(z, axis=0, keepdims=True)


def _pool(x1, x2, x3, w, b):
    full = lambda a: pl.BlockSpec(a.shape, lambda f: (0,) * a.ndim)
    wa, wb, wc = w[:OUT_D], w[OUT_D:2 * OUT_D], w[2 * OUT_D:]
    xspec = pl.BlockSpec((1, N, OUT_D), lambda f: (f, 0, 0))
    return pl.pallas_call(
        _pool_body,
        grid=(FRAMES,),
        in_specs=[xspec, xspec, xspec, full(wa), full(wb), full(wc),
                  full(b.reshape(1, DM))],
        out_specs=pl.BlockSpec((1, 1, DM), lambda f: (f, 0, 0)),
        out_shape=jax.ShapeDtypeStruct((FRAMES, 1, DM), _F32),
    )(x1, x2, x3, wa, wb, wc, b.reshape(1, DM)).reshape(FRAMES, DM)


def _ln2(x, g, b):
    mu = jnp.mean(x, axis=1, keepdims=True)
    var = jnp.mean((x - mu) * (x - mu), axis=1, keepdims=True)
    return (x - mu) * jax.lax.rsqrt(var + 1e-5) * g + b


def _tf_head_body(seq_ref, wq_ref, bq_ref, wk_ref, bk_ref, wv_ref, bv_ref,
                  wo_ref, bo_ref, l1g_ref, l1b_ref, w1_ref, b1_ref,
                  w2_ref, b2_ref, l2g_ref, l2b_ref,
                  hw0_ref, hb0_ref, hw1_ref, hb1_ref, hw2_ref, hb2_ref,
                  hw3_ref, hb3_ref, out_ref):
    seq = seq_ref[...]
    q = _dot(seq, wq_ref[...]) + bq_ref[...]
    k = _dot(seq, wk_ref[...]) + bk_ref[...]
    v = _dot(seq, wv_ref[...]) + bv_ref[...]
    ii = jax.lax.broadcasted_iota(jnp.int32, (FRAMES, FRAMES), 0) // T
    jj = jax.lax.broadcasted_iota(jnp.int32, (FRAMES, FRAMES), 1) // T
    blk = ii == jj
    ctxs = []
    for h in range(NH):
        sl = slice(h * DH, (h + 1) * DH)
        s = _dot_t(q[:, sl], k[:, sl]) * (1.0 / 16.0)
        s = jnp.where(blk, s, -1e30)
        s = s - jnp.max(s, axis=1, keepdims=True)
        e = jnp.exp(s)
        att = e / jnp.sum(e, axis=1, keepdims=True)
        ctxs.append(_dot(att, v[:, sl]))
    ctx = jnp.concatenate(ctxs, axis=1)
    y = _ln2(seq + _dot(ctx, wo_ref[...]) + bo_ref[...],
             l1g_ref[...], l1b_ref[...])
    ff = _dot(jax.nn.relu(_dot(y, w1_ref[...]) + b1_ref[...]),
              w2_ref[...]) + b2_ref[...]
    y = _ln2(y + ff, l2g_ref[...], l2b_ref[...])
    feat = jnp.mean(y.reshape(B, T, DM), axis=1)
    h0 = jax.nn.relu(_dot(feat, hw0_ref[...]) + hb0_ref[...])
    h1 = jax.nn.relu(_dot(h0, hw1_ref[...]) + hb1_ref[...])
    h2 = jax.nn.relu(_dot(h1, hw2_ref[...]) + hb2_ref[...])
    out_ref[...] = _dot(h2, hw3_ref[...]) + hb3_ref[...]


def _tf_head(seq, tf, head):
    row = lambda v: v.reshape(1, -1)
    args = (seq, tf['Wq'], row(tf['bq']), tf['Wk'], row(tf['bk']),
            tf['Wv'], row(tf['bv']), tf['Wo'], row(tf['bo']),
            row(tf['ln1_g']), row(tf['ln1_b']), tf['W1'], row(tf['b1']),
            tf['W2'], row(tf['b2']), row(tf['ln2_g']), row(tf['ln2_b']),
            head[0]['W'], row(head[0]['b']), head[1]['W'], row(head[1]['b']),
            head[2]['W'], row(head[2]['b']), head[3]['W'], row(head[3]['b']))
    return pl.pallas_call(
        _tf_head_body,
        out_shape=jax.ShapeDtypeStruct((B, NC), _F32),
    )(*args)


def kernel(data, params):
    geom = data[..., :GEOM].reshape(FRAMES, N, GEOM)
    aux = data[..., GEOM:GEOM + AUX].reshape(FRAMES, N, AUX)
    x1 = _edge_layer(geom, aux, params['edge'][0])
    x2 = _edge_layer(x1, aux, params['edge'][1])
    x3 = _edge_layer(x2, aux, params['edge'][2])
    pooled = _pool(x1, x2, x3, params['lin1_W'], params['lin1_b'])
    pos = jnp.tile(params['pos'][0, :T, :], (B, 1))
    seq = pooled + pos
    return _tf_head(seq, params['tf'], params['head'])


# speedup vs baseline: 19.4699x; 1.6349x over previous
import functools

import jax
import jax.numpy as jnp
from jax.experimental import pallas as pl

B, T, N, C = 4, 16, 512, 7
GEOM, AUX, K = 3, 4, 20
FRAMES = B * T
OUT_D = 32
DM, NH, DH, FF, NC = 1024, 4, 256, 2048, 12

_F32 = jnp.float32
_IMAX = 2147483647


def _dot(a, b):
    return jax.lax.dot_general(a, b, (((1,), (0,)), ((), ())),
                               preferred_element_type=_F32)


def _dot_t(a, b):
    return jax.lax.dot_general(a, b, (((1,), (1,)), ((), ())),
                               preferred_element_type=_F32)


def _edge_layer_body(in_d, x_ref, aux_ref, wl_ref, wr_ref, eb1_ref,
                     ew2_ref, eb2_ref, a1a_ref, a1b_ref, ab1_ref,
                     aw2_ref, ab2_ref, lng_ref, lnb_ref, out_ref):
    g = x_ref[0]
    aux = aux_ref[0]

    gg = g * g
    sq_col = jnp.sum(gg, axis=1, keepdims=True)
    sq_row = _dot_t(jnp.ones((1, in_d), _F32), gg)
    d2 = jnp.maximum(sq_col + sq_row - 2.0 * _dot_t(g, g), 0.0)
    ii = jax.lax.broadcasted_iota(jnp.int32, (N, N), 0)
    jj = jax.lax.broadcasted_iota(jnp.int32, (N, N), 1)
    d2 = d2 + jnp.where(ii == jj, 1e10, 0.0).astype(_F32)
    key = jnp.bitwise_or(
        jnp.bitwise_and(jax.lax.bitcast_convert_type(d2, jnp.int32),
                        -512), ii)

    p = _dot(g, wl_ref[...]) + eb1_ref[...]
    q = _dot(g, wr_ref[...])
    at = _dot(aux, a1a_ref[...]) + ab1_ref[...]
    asrc = _dot(aux, a1b_ref[...])
    payload = jnp.concatenate([q, asrc], axis=1)

    gaths = []
    for _ in range(K):
        m = jnp.min(key, axis=0, keepdims=True)
        sel = key == m
        onehot_t = sel.astype(_F32)
        key = jnp.where(sel, _IMAX, key)
        gaths.append(jax.lax.dot_general(
            onehot_t, payload, (((0,), (0,)), ((), ())),
            preferred_element_type=_F32))
    gath = jnp.concatenate(gaths, axis=0)
    h1 = jax.nn.relu(jnp.tile(p, (K, 1)) + gath[:, :OUT_D])
    a1 = jax.nn.relu(jnp.tile(at, (K, 1)) + gath[:, OUT_D:])
    h2 = jax.nn.relu(_dot(h1, ew2_ref[...]) + eb2_ref[...])
    gb = _dot(a1, aw2_ref[...]) + ab2_ref[...]
    mod = (jax.nn.sigmoid(gb[:, :OUT_D] + 1.0) * h2 + gb[:, OUT_D:])
    acc = jnp.max(mod.reshape(K, N, OUT_D), axis=0)

    mu = jnp.mean(acc, axis=1, keepdims=True)
    var = jnp.mean((acc - mu) * (acc - mu), axis=1, keepdims=True)
    y = (acc - mu) * jax.lax.rsqrt(var + 1e-5) * lng_ref[...] + lnb_ref[...]
    out_ref[0] = jax.nn.relu(y)


def _edge_layer(x, aux, lp):
    in_d = x.shape[-1]
    wl = lp['eW1'][:in_d] - lp['eW1'][in_d:]
    wr = lp['eW1'][in_d:]
    a1a = lp['aW1'][:AUX]
    a1b = lp['aW1'][AUX:]
    row = lambda v: v.reshape(1, -1)
    full = lambda a: pl.BlockSpec(a.shape, lambda f: (0,) * a.ndim)
    args = (aux, wl, wr, row(lp['eb1']), lp['eW2'], row(lp['eb2']),
            a1a, a1b, row(lp['ab1']), lp['aW2'], row(lp['ab2']),
            row(lp['ln_g']), row(lp['ln_b']))
    return pl.pallas_call(
        functools.partial(_edge_layer_body, in_d),
        grid=(FRAMES,),
        in_specs=[pl.BlockSpec((1, N, in_d), lambda f: (f, 0, 0)),
                  pl.BlockSpec((1, N, AUX), lambda f: (f, 0, 0))]
                 + [full(a) for a in args[1:]],
        out_specs=pl.BlockSpec((1, N, OUT_D), lambda f: (f, 0, 0)),
        out_shape=jax.ShapeDtypeStruct((FRAMES, N, OUT_D), _F32),
    )(x, *args)


def _pool_body(x1_ref, x2_ref, x3_ref, wa_ref, wb_ref, wc_ref, b_ref,
               out_ref):
    z = (_dot(x1_ref[0], wa_ref[...]) + _dot(x2_ref[0], wb_ref[...])
         + _dot(x3_ref[0], wc_ref[...]) + b_ref[...])
    z = jax.nn.relu(z)
    out_ref[0] = jnp.max(z, axis=0, keepdims=True)


def _pool(x1, x2, x3, w, b):
    full = lambda a: pl.BlockSpec(a.shape, lambda f: (0,) * a.ndim)
    wa, wb, wc = w[:OUT_D], w[OUT_D:2 * OUT_D], w[2 * OUT_D:]
    xspec = pl.BlockSpec((1, N, OUT_D), lambda f: (f, 0, 0))
    return pl.pallas_call(
        _pool_body,
        grid=(FRAMES,),
        in_specs=[xspec, xspec, xspec, full(wa), full(wb), full(wc),
                  full(b.reshape(1, DM))],
        out_specs=pl.BlockSpec((1, 1, DM), lambda f: (f, 0, 0)),
        out_shape=jax.ShapeDtypeStruct((FRAMES, 1, DM), _F32),
    )(x1, x2, x3, wa, wb, wc, b.reshape(1, DM)).reshape(FRAMES, DM)


def _ln2(x, g, b):
    mu = jnp.mean(x, axis=1, keepdims=True)
    var = jnp.mean((x - mu) * (x - mu), axis=1, keepdims=True)
    return (x - mu) * jax.lax.rsqrt(var + 1e-5) * g + b


def _tf_head_body(seq_ref, wq_ref, bq_ref, wk_ref, bk_ref, wv_ref, bv_ref,
                  wo_ref, bo_ref, l1g_ref, l1b_ref, w1_ref, b1_ref,
                  w2_ref, b2_ref, l2g_ref, l2b_ref,
                  hw0_ref, hb0_ref, hw1_ref, hb1_ref, hw2_ref, hb2_ref,
                  hw3_ref, hb3_ref, out_ref):
    seq = seq_ref[...]
    q = _dot(seq, wq_ref[...]) + bq_ref[...]
    k = _dot(seq, wk_ref[...]) + bk_ref[...]
    v = _dot(seq, wv_ref[...]) + bv_ref[...]
    ii = jax.lax.broadcasted_iota(jnp.int32, (FRAMES, FRAMES), 0) // T
    jj = jax.lax.broadcasted_iota(jnp.int32, (FRAMES, FRAMES), 1) // T
    blk = ii == jj
    ctxs = []
    for h in range(NH):
        sl = slice(h * DH, (h + 1) * DH)
        s = _dot_t(q[:, sl], k[:, sl]) * (1.0 / 16.0)
        s = jnp.where(blk, s, -1e30)
        s = s - jnp.max(s, axis=1, keepdims=True)
        e = jnp.exp(s)
        att = e / jnp.sum(e, axis=1, keepdims=True)
        ctxs.append(_dot(att, v[:, sl]))
    ctx = jnp.concatenate(ctxs, axis=1)
    y = _ln2(seq + _dot(ctx, wo_ref[...]) + bo_ref[...],
             l1g_ref[...], l1b_ref[...])
    ff = _dot(jax.nn.relu(_dot(y, w1_ref[...]) + b1_ref[...]),
              w2_ref[...]) + b2_ref[...]
    y = _ln2(y + ff, l2g_ref[...], l2b_ref[...])
    feat = jnp.mean(y.reshape(B, T, DM), axis=1)
    h0 = jax.nn.relu(_dot(feat, hw0_ref[...]) + hb0_ref[...])
    h1 = jax.nn.relu(_dot(h0, hw1_ref[...]) + hb1_ref[...])
    h2 = jax.nn.relu(_dot(h1, hw2_ref[...]) + hb2_ref[...])
    out_ref[...] = _dot(h2, hw3_ref[...]) + hb3_ref[...]


def _tf_head(seq, tf, head):
    row = lambda v: v.reshape(1, -1)
    args = (seq, tf['Wq'], row(tf['bq']), tf['Wk'], row(tf['bk']),
            tf['Wv'], row(tf['bv']), tf['Wo'], row(tf['bo']),
            row(tf['ln1_g']), row(tf['ln1_b']), tf['W1'], row(tf['b1']),
            tf['W2'], row(tf['b2']), row(tf['ln2_g']), row(tf['ln2_b']),
            head[0]['W'], row(head[0]['b']), head[1]['W'], row(head[1]['b']),
            head[2]['W'], row(head[2]['b']), head[3]['W'], row(head[3]['b']))
    return pl.pallas_call(
        _tf_head_body,
        out_shape=jax.ShapeDtypeStruct((B, NC), _F32),
    )(*args)


def kernel(data, params):
    geom = data[..., :GEOM].reshape(FRAMES, N, GEOM)
    aux = data[..., GEOM:GEOM + AUX].reshape(FRAMES, N, AUX)
    x1 = _edge_layer(geom, aux, params['edge'][0])
    x2 = _edge_layer(x1, aux, params['edge'][1])
    x3 = _edge_layer(x2, aux, params['edge'][2])
    pooled = _pool(x1, x2, x3, params['lin1_W'], params['lin1_b'])
    pos = jnp.tile(params['pos'][0, :T, :], (B, 1))
    seq = pooled + pos
    return _tf_head(seq, params['tf'], params['head'])
